# Initial kernel scaffold; baseline (speedup 1.0000x reference)
import functools

import jax
import jax.numpy as jnp
from jax import lax
from jax.experimental import pallas as pl
from jax.experimental.pallas import tpu as pltpu
from jax.experimental.pallas import tpu_sc as plsc

N_ROWS = 10000
D = 128
E = 320000

NC = 2
NS = 16
NW = NC * NS

EW = E // NW
CB = 80
NCHUNK = EW // CB


def _dot_chunk(ra, rb, out_v, b, out_base):
    def edge_group(i, _):
        for j in range(4):
            e = i * 4 + j
            prods = []
            for k in range(8):
                va = ra[b, e, pl.ds(k * 16, 16)]
                vb = rb[b, e, pl.ds(k * 16, 16)]
                prods.append(va * vb)
            s01 = (prods[0] + prods[1]) + (prods[2] + prods[3])
            s23 = (prods[4] + prods[5]) + (prods[6] + prods[7])
            acc = s01 + s23
            out_v[out_base + e] = jnp.sum(acc)
        return 0

    lax.fori_loop(0, CB // 4, edge_group, 0, unroll=False)


def _kernel_body(xa_hbm, xb_hbm, edge_hbm, out_hbm,
                 ia, ib, ra, rb, out_v, sem0, sem1):
    cid = lax.axis_index("c")
    sid = lax.axis_index("s")
    wid = sid * NC + cid
    wbase = wid * EW

    sems = (sem0, sem1)

    def prefetch(c, b):
        base = wbase + c * CB
        pltpu.sync_copy(edge_hbm.at[0, pl.ds(base, CB)], ia.at[b])
        pltpu.sync_copy(edge_hbm.at[1, pl.ds(base, CB)], ib.at[b])
        pltpu.make_async_copy(xa_hbm.at[ia.at[b]], ra.at[b], sems[b]).start()
        pltpu.make_async_copy(xb_hbm.at[ib.at[b]], rb.at[b], sems[b]).start()

    def wait_chunk(b):
        pltpu.make_async_copy(xa_hbm.at[ia.at[b]], ra.at[b], sems[b]).wait()
        pltpu.make_async_copy(xb_hbm.at[ib.at[b]], rb.at[b], sems[b]).wait()

    prefetch(0, 0)

    def step(i, _):
        c_base = i * 2
        for b in (0, 1):
            c = c_base + b
            nxt = c + 1

            @pl.when(nxt < NCHUNK)
            def _():
                prefetch(nxt, 1 - b)

            @pl.when(c < NCHUNK)
            def _():
                wait_chunk(b)
                _dot_chunk(ra, rb, out_v, b, c * CB)

        return 0

    lax.fori_loop(0, (NCHUNK + 1) // 2, step, 0, unroll=False)

    pltpu.sync_copy(out_v, out_hbm.at[pl.ds(wbase, EW)])


@jax.jit
def _run(xa, xb, edges):
    mesh = plsc.VectorSubcoreMesh(core_axis_name="c", subcore_axis_name="s")
    return pl.kernel(
        _kernel_body,
        out_type=jax.ShapeDtypeStruct((E,), jnp.float32),
        mesh=mesh,
        scratch_types=[
            pltpu.VMEM((2, CB), jnp.int32),
            pltpu.VMEM((2, CB), jnp.int32),
            pltpu.VMEM((2, CB, D), jnp.float32),
            pltpu.VMEM((2, CB, D), jnp.float32),
            pltpu.VMEM((EW,), jnp.float32),
            pltpu.SemaphoreType.DMA,
            pltpu.SemaphoreType.DMA,
        ],
    )(xa, xb, edges)


def kernel(x_miRNA, x_disease, edge_label_index):
    edges = edge_label_index.astype(jnp.int32)
    return _run(x_miRNA, x_disease, edges)


# speedup vs baseline: 9.0333x; 9.0333x over previous
import jax
import jax.numpy as jnp
from jax import lax
from jax.experimental import pallas as pl
from jax.experimental.pallas import tpu as pltpu
from jax.experimental.pallas import tpu_sc as plsc

N_ROWS = 10000
D = 128
E = 320000

NC = 2
NS = 16
NW = NC * NS

EW = E // NW
CB = 80
NCHUNK = EW // CB


def _dot_chunk(ra, rb, out_v, b, out_base):
    lanes = lax.iota(jnp.int32, 16)

    def group(g, _):
        gbase = g * 16

        def quad(m, out16):
            for jj in range(4):
                j = m * 4 + jj
                e = gbase + j
                prods = []
                for k in range(8):
                    va = ra[b, e, pl.ds(k * 16, 16)]
                    vb = rb[b, e, pl.ds(k * 16, 16)]
                    prods.append(va * vb)
                s01 = (prods[0] + prods[1]) + (prods[2] + prods[3])
                s23 = (prods[4] + prods[5]) + (prods[6] + prods[7])
                s = jnp.sum(s01 + s23)
                out16 = jnp.where(lanes == j, s, out16)
            return out16

        out16 = lax.fori_loop(0, 4, quad, jnp.zeros((16,), jnp.float32),
                              unroll=False)
        out_v[pl.ds(out_base + gbase, 16)] = out16
        return 0

    lax.fori_loop(0, CB // 16, group, 0, unroll=False)


def _kernel_body(xa_hbm, xb_hbm, src_hbm, dst_hbm, out_hbm,
                 ia, ib, ra, rb, out_v, sem0, sem1):
    cid = lax.axis_index("c")
    sid = lax.axis_index("s")
    wid = sid * NC + cid
    wbase = wid * EW

    sems = (sem0, sem1)

    pltpu.sync_copy(src_hbm.at[pl.ds(wbase, EW)], ia)
    pltpu.sync_copy(dst_hbm.at[pl.ds(wbase, EW)], ib)

    def gather(c, b):
        off = c * CB
        pltpu.make_async_copy(
            xa_hbm.at[ia.at[pl.ds(off, CB)]], ra.at[b], sems[b]).start()
        pltpu.make_async_copy(
            xb_hbm.at[ib.at[pl.ds(off, CB)]], rb.at[b], sems[b]).start()

    def wait_chunk(b):
        pltpu.make_async_copy(xa_hbm.at[ia.at[pl.ds(0, CB)]],
                              ra.at[b], sems[b]).wait()
        pltpu.make_async_copy(xb_hbm.at[ib.at[pl.ds(0, CB)]],
                              rb.at[b], sems[b]).wait()

    gather(0, 0)

    def step(i, _):
        c_base = i * 2
        for b in (0, 1):
            c = c_base + b
            nxt = c + 1

            @pl.when(nxt < NCHUNK)
            def _():
                gather(nxt, 1 - b)

            @pl.when(c < NCHUNK)
            def _():
                wait_chunk(b)
                _dot_chunk(ra, rb, out_v, b, c * CB)

        return 0

    lax.fori_loop(0, (NCHUNK + 1) // 2, step, 0, unroll=False)

    pltpu.sync_copy(out_v, out_hbm.at[pl.ds(wbase, EW)])


@jax.jit
def _run(xa, xb, src, dst):
    mesh = plsc.VectorSubcoreMesh(core_axis_name="c", subcore_axis_name="s")
    return pl.kernel(
        _kernel_body,
        out_type=jax.ShapeDtypeStruct((E,), jnp.float32),
        mesh=mesh,
        compiler_params=pltpu.CompilerParams(needs_layout_passes=False),
        scratch_types=[
            pltpu.VMEM((EW,), jnp.int32),
            pltpu.VMEM((EW,), jnp.int32),
            pltpu.VMEM((2, CB, D), jnp.float32),
            pltpu.VMEM((2, CB, D), jnp.float32),
            pltpu.VMEM((EW,), jnp.float32),
            pltpu.SemaphoreType.DMA,
            pltpu.SemaphoreType.DMA,
        ],
    )(xa, xb, src, dst)


def kernel(x_miRNA, x_disease, edge_label_index):
    edges = edge_label_index.astype(jnp.int32)
    return _run(x_miRNA, x_disease, edges[0], edges[1])
